# Initial kernel scaffold; baseline (speedup 1.0000x reference)
import jax
import jax.numpy as jnp
from jax.experimental import pallas as pl


def kernel(edge_rep, face_rep, edge_index, face_to_message_indicator, W1, W2, W3, g1, b1, W4, g2, b2, W5, g3, b3, W6, g4, b4, W7, b7):
    raise NotImplementedError("write your pallas kernel here")


# speedup vs baseline: 1.9264x; 1.9264x over previous
import functools

import jax
import jax.numpy as jnp
from jax import lax
from jax.experimental import pallas as pl
from jax.experimental.pallas import tpu as pltpu
from jax.experimental.pallas import tpu_sc as plsc

N = 10000
F = 20000
E = 320000
H = 128
H2 = 256
EPS = 1e-5

NC = 2
NS = 16
NW = NC * NS
EW = E // NW
K = 80
NCHUNK = EW // K
NROW = N // NS


def _proj2_body(x_ref, w1_ref, w2_ref, a_ref, b_ref):
    x = x_ref[...]
    dn = (((1,), (1,)), ((), ()))
    a_ref[...] = lax.dot_general(x, w1_ref[...], dn,
                                 preferred_element_type=jnp.float32)
    b_ref[...] = lax.dot_general(x, w2_ref[...], dn,
                                 preferred_element_type=jnp.float32)


def _proj1_body(x_ref, w_ref, o_ref):
    o_ref[...] = lax.dot_general(x_ref[...], w_ref[...], (((1,), (1,)), ((), ())),
                                 preferred_element_type=jnp.float32)


def _sc_gather_body(a_hbm, b_hbm, c_hbm, src_hbm, dst_hbm, f2m_hbm,
                    t_hbm, stats_hbm,
                    src_v, dst_v, f2m_v, ar, br, cr, tb, acc, sem):
    cid = lax.axis_index("c")
    sid = lax.axis_index("s")
    wid = sid * NC + cid

    pltpu.sync_copy(src_hbm.at[wid], src_v)
    pltpu.sync_copy(dst_hbm.at[wid], dst_v)
    pltpu.sync_copy(f2m_hbm.at[wid], f2m_v)

    for g in range(H2 // 16):
        acc[0, pl.ds(g * 16, 16)] = jnp.zeros((16,), jnp.float32)
        acc[1, pl.ds(g * 16, 16)] = jnp.zeros((16,), jnp.float32)

    def chunk(c, _):
        d1 = pltpu.async_copy(a_hbm.at[src_v.at[pl.ds(c * K, K)]], ar, sem)
        d2 = pltpu.async_copy(b_hbm.at[dst_v.at[pl.ds(c * K, K)]], br, sem)
        d3 = pltpu.async_copy(c_hbm.at[f2m_v.at[pl.ds(c * K, K)]], cr, sem)
        d1.wait()
        d2.wait()
        d3.wait()
        for g in range(H2 // 16):
            gs = pl.ds(g * 16, 16)

            def row(r, carry):
                s, q = carry
                t = ar[r, gs] + br[r, gs] + cr[r, gs]
                tb[r, gs] = t
                return s + t, q + t * t

            s, q = lax.fori_loop(0, K, row,
                                 (jnp.zeros((16,), jnp.float32),
                                  jnp.zeros((16,), jnp.float32)))
            acc[0, gs] = acc[0, gs] + s
            acc[1, gs] = acc[1, gs] + q
        pltpu.sync_copy(tb, t_hbm.at[pl.ds(wid * EW + c * K, K)])
        return _

    lax.fori_loop(0, NCHUNK, chunk, 0)
    pltpu.sync_copy(acc, stats_hbm.at[wid])


def _bn1_body(t_ref, a1_ref, b1_ref, w4_ref, m2_ref, mean2_ref, msq2_ref,
              s_acc, xs_acc):
    i = pl.program_id(0)

    @pl.when(i == 0)
    def _():
        s_acc[...] = jnp.zeros_like(s_acc)
        xs_acc[...] = jnp.zeros_like(xs_acc)

    x = jnp.maximum(t_ref[...] * a1_ref[...] + b1_ref[...], 0.0)
    s_acc[...] += lax.dot_general(x, x, (((0,), (0,)), ((), ())),
                                  preferred_element_type=jnp.float32)
    xs_acc[...] += jnp.sum(x, axis=0, keepdims=True)
    w4 = w4_ref[...]
    m2_ref[...] = lax.dot_general(x, w4, (((1,), (1,)), ((), ())),
                                  preferred_element_type=jnp.float32)

    @pl.when(i == pl.num_programs(0) - 1)
    def _():
        inv_e = 1.0 / E
        mean2_ref[...] = lax.dot_general(
            xs_acc[...], w4, (((1,), (1,)), ((), ())),
            preferred_element_type=jnp.float32) * inv_e
        ws = lax.dot_general(w4, s_acc[...], (((1,), (0,)), ((), ())),
                             preferred_element_type=jnp.float32)
        msq2_ref[...] = jnp.sum(ws * w4, axis=1)[None, :] * inv_e


NYCHUNK = N // K


def _sc_scatter_body(m2_hbm, dst_hbm, s2_hbm, sh2_hbm,
                     y_hbm,
                     idx2d, mr, s2_v, sh2_v, y_sh, sem):
    cid = lax.axis_index("c")
    sid = lax.axis_index("s")
    wid = sid * NC + cid

    pltpu.sync_copy(s2_hbm, s2_v)
    pltpu.sync_copy(sh2_hbm, sh2_v)
    pltpu.sync_copy(dst_hbm.at[wid], idx2d)

    for g in range(H // 16):
        def zrow(r, _):
            mr[r, pl.ds(g * 16, 16)] = jnp.zeros((16,), jnp.float32)
            return _
        lax.fori_loop(0, K, zrow, 0)

    nz = (NYCHUNK - sid + NS - 1) // NS

    def zchunk(j, _):
        pltpu.sync_copy(mr, y_sh.at[pl.ds((sid + j * NS) * K, K)])
        return _

    lax.fori_loop(0, nz, zchunk, 0)
    plsc.subcore_barrier()

    def chunk(c, _):
        pltpu.async_copy(m2_hbm.at[pl.ds(wid * EW + c * K, K)], mr, sem).wait()
        for g in range(H // 16):
            gs = pl.ds(g * 16, 16)
            s2 = s2_v[gs]
            sh2 = sh2_v[gs]

            def row(r, _):
                mr[r, gs] = jnp.maximum(mr[r, gs] * s2 + sh2, 0.0)
                return _

            lax.fori_loop(0, K, row, 0)
        pltpu.sync_copy(mr, y_sh.at[idx2d.at[c]], add=True)
        return _

    lax.fori_loop(0, NCHUNK, chunk, 0)
    plsc.subcore_barrier()

    def wchunk(j, _):
        c = sid + j * NS
        pltpu.sync_copy(y_sh.at[pl.ds(c * K, K)],
                        y_hbm.at[cid, pl.ds(c * K, K)])
        return _

    lax.fori_loop(0, nz, wchunk, 0)


def _mlp2_body(y2_ref, w5_ref, g3_ref, b3_ref, w6_ref, g4_ref, b4_ref,
               w7_ref, b7_ref, o_ref):
    dn = (((1,), (1,)), ((), ()))
    y = y2_ref[0] + y2_ref[1]
    z = lax.dot_general(y, w5_ref[...], dn, preferred_element_type=jnp.float32)
    mu = jnp.mean(z, axis=0, keepdims=True)
    v = jnp.mean((z - mu) ** 2, axis=0, keepdims=True)
    z = jnp.maximum((z - mu) * (g3_ref[...] * lax.rsqrt(v + EPS)) + b3_ref[...], 0.0)
    w = lax.dot_general(z, w6_ref[...], dn, preferred_element_type=jnp.float32)
    mu = jnp.mean(w, axis=0, keepdims=True)
    v = jnp.mean((w - mu) ** 2, axis=0, keepdims=True)
    w = jnp.maximum((w - mu) * (g4_ref[...] * lax.rsqrt(v + EPS)) + b4_ref[...], 0.0)
    o_ref[...] = lax.dot_general(w, w7_ref[...], dn,
                                 preferred_element_type=jnp.float32) + b7_ref[...]


def kernel(edge_rep, face_rep, edge_index, face_to_message_indicator,
           W1, W2, W3, g1, b1, W4, g2, b2, W5, g3, b3, W6, g4, b4, W7, b7):
    f32 = jnp.float32
    src = edge_index[0].astype(jnp.int32).reshape(NW, EW)
    dst = edge_index[1].astype(jnp.int32).reshape(NW, EW)
    f2m = face_to_message_indicator.astype(jnp.int32).reshape(NW, EW)
    dst3 = edge_index[1].astype(jnp.int32).reshape(NW, NCHUNK, K)

    A, B = pl.pallas_call(
        _proj2_body,
        out_shape=(jax.ShapeDtypeStruct((N, H2), f32),
                   jax.ShapeDtypeStruct((N, H2), f32)),
    )(edge_rep, W1, W2)
    C = pl.pallas_call(
        _proj1_body,
        out_shape=jax.ShapeDtypeStruct((F, H2), f32),
    )(face_rep, W3)

    mesh = plsc.VectorSubcoreMesh(core_axis_name="c", subcore_axis_name="s",
                                  num_cores=NC, num_subcores=NS)
    t, stats = pl.kernel(
        _sc_gather_body,
        out_type=(jax.ShapeDtypeStruct((E, H2), f32),
                  jax.ShapeDtypeStruct((NW, 2, H2), f32)),
        mesh=mesh,
        scratch_types=[
            pltpu.VMEM((EW,), jnp.int32),
            pltpu.VMEM((EW,), jnp.int32),
            pltpu.VMEM((EW,), jnp.int32),
            pltpu.VMEM((K, H2), f32),
            pltpu.VMEM((K, H2), f32),
            pltpu.VMEM((K, H2), f32),
            pltpu.VMEM((K, H2), f32),
            pltpu.VMEM((2, H2), f32),
            pltpu.SemaphoreType.DMA,
        ],
    )(A, B, C, src, dst, f2m)

    st = jnp.sum(stats, axis=0)
    mu1 = st[0] / E
    var1 = st[1] / E - mu1 * mu1
    a1 = (g1 * lax.rsqrt(var1 + EPS))[None, :]
    b1p = (b1 - mu1 * a1[0])[None, :]

    BE = 3200
    m2, mean2, msq2 = pl.pallas_call(
        _bn1_body,
        grid=(E // BE,),
        in_specs=[
            pl.BlockSpec((BE, H2), lambda i: (i, 0)),
            pl.BlockSpec((1, H2), lambda i: (0, 0)),
            pl.BlockSpec((1, H2), lambda i: (0, 0)),
            pl.BlockSpec((H, H2), lambda i: (0, 0)),
        ],
        out_specs=[
            pl.BlockSpec((BE, H), lambda i: (i, 0)),
            pl.BlockSpec((1, H), lambda i: (0, 0)),
            pl.BlockSpec((1, H), lambda i: (0, 0)),
        ],
        out_shape=[
            jax.ShapeDtypeStruct((E, H), f32),
            jax.ShapeDtypeStruct((1, H), f32),
            jax.ShapeDtypeStruct((1, H), f32),
        ],
        scratch_shapes=[
            pltpu.VMEM((H2, H2), f32),
            pltpu.VMEM((1, H2), f32),
        ],
    )(t, a1, b1p, W4)

    var2 = msq2[0] - mean2[0] ** 2
    s2 = g2 * lax.rsqrt(var2 + EPS)
    sh2 = b2 - mean2[0] * s2

    y2 = pl.kernel(
        _sc_scatter_body,
        out_type=jax.ShapeDtypeStruct((NC, N, H), f32),
        mesh=mesh,
        scratch_types=[
            pltpu.VMEM((NCHUNK, K), jnp.int32),
            pltpu.VMEM((K, H), f32),
            pltpu.VMEM((H,), f32),
            pltpu.VMEM((H,), f32),
            pltpu.VMEM_SHARED((N, H), f32),
            pltpu.SemaphoreType.DMA,
        ],
    )(m2, dst3, s2, sh2)

    out = pl.pallas_call(
        _mlp2_body,
        out_shape=jax.ShapeDtypeStruct((N, H), f32),
    )(y2, W5, g3[None, :], b3[None, :], W6, g4[None, :], b4[None, :],
      W7, b7[None, :])
    return out


# speedup vs baseline: 3.2993x; 1.7127x over previous
import functools

import jax
import jax.numpy as jnp
from jax import lax
from jax.experimental import pallas as pl
from jax.experimental.pallas import tpu as pltpu
from jax.experimental.pallas import tpu_sc as plsc

N = 10000
F = 20000
E = 320000
H = 128
H2 = 256
EPS = 1e-5

NC = 2
NS = 16
NW = NC * NS
EW = E // NW

KG = 40
NCG = EW // KG
KS = 80
NCS = EW // KS
NG2 = H2 // 16
NG1 = H // 16


def _proj2_body(x_ref, w1_ref, w2_ref, a_ref, b_ref):
    x = x_ref[...]
    dn = (((1,), (1,)), ((), ()))
    a_ref[...] = lax.dot_general(x, w1_ref[...], dn,
                                 preferred_element_type=jnp.float32)
    b_ref[...] = lax.dot_general(x, w2_ref[...], dn,
                                 preferred_element_type=jnp.float32)


def _proj1_body(x_ref, w_ref, o_ref):
    o_ref[...] = lax.dot_general(x_ref[...], w_ref[...], (((1,), (1,)), ((), ())),
                                 preferred_element_type=jnp.float32)


def _sc_gather_body(a_hbm, b_hbm, c_hbm, src_hbm, dst_hbm, f2m_hbm,
                    t_hbm, stats_hbm,
                    src_v, dst_v, f2m_v,
                    ar0, br0, cr0, tb0, ar1, br1, cr1, tb1,
                    acc, gs0, gs1, ws0, ws1):
    cid = lax.axis_index("c")
    sid = lax.axis_index("s")
    wid = sid * NC + cid

    pltpu.sync_copy(src_hbm.at[wid], src_v)
    pltpu.sync_copy(dst_hbm.at[wid], dst_v)
    pltpu.sync_copy(f2m_hbm.at[wid], f2m_v)

    for g in range(NG2):
        acc[0, pl.ds(g * 16, 16)] = jnp.zeros((16,), jnp.float32)
        acc[1, pl.ds(g * 16, 16)] = jnp.zeros((16,), jnp.float32)

    bufs = ((ar0, br0, cr0, tb0, gs0, ws0), (ar1, br1, cr1, tb1, gs1, ws1))

    def issue(c, b):
        ar, br, cr, _, gs, _ = bufs[b]
        sl = pl.ds(c * KG, KG)
        pltpu.async_copy(a_hbm.at[src_v.at[sl]], ar, gs)
        pltpu.async_copy(b_hbm.at[dst_v.at[sl]], br, gs)
        pltpu.async_copy(c_hbm.at[f2m_v.at[sl]], cr, gs)

    issue(0, 0)
    issue(1, 1)

    zero16 = jnp.zeros((16,), jnp.float32)

    def outer(i2, carry_unused):
        for b in range(2):
            c = i2 * 2 + b
            ar, br, cr, tb, gs, ws = bufs[b]
            sl = pl.ds(c * KG, KG)
            pltpu.make_async_copy(a_hbm.at[src_v.at[sl]], ar, gs).wait()
            pltpu.make_async_copy(b_hbm.at[dst_v.at[sl]], br, gs).wait()
            pltpu.make_async_copy(c_hbm.at[f2m_v.at[sl]], cr, gs).wait()

            @pl.when(c >= 2)
            def _():
                pltpu.make_async_copy(
                    tb, t_hbm.at[pl.ds(wid * EW + (c - 2) * KG, KG)], ws).wait()

            def row(r, carry):
                out = []
                for g in range(NG2):
                    gsl = pl.ds(g * 16, 16)
                    t = ar[r, gsl] + br[r, gsl] + cr[r, gsl]
                    tb[r, gsl] = t
                    out.append(carry[g] + t)
                    out.append(carry[NG2 + g] + t * t)
                return tuple(out[0::2]) + tuple(out[1::2])

            sums = lax.fori_loop(0, KG, row, (zero16,) * (2 * NG2))
            for g in range(NG2):
                gsl = pl.ds(g * 16, 16)
                acc[0, gsl] = acc[0, gsl] + sums[g]
                acc[1, gsl] = acc[1, gsl] + sums[NG2 + g]

            pltpu.async_copy(tb, t_hbm.at[pl.ds(wid * EW + c * KG, KG)], ws)

            @pl.when(c + 2 < NCG)
            def _():
                issue(c + 2, b)
        return carry_unused

    lax.fori_loop(0, NCG // 2, outer, 0)
    for b in range(2):
        tb = bufs[b][3]
        ws = bufs[b][5]
        pltpu.make_async_copy(
            tb, t_hbm.at[pl.ds(wid * EW + (NCG - 2 + b) * KG, KG)], ws).wait()
    pltpu.sync_copy(acc, stats_hbm.at[wid])


def _bn1_body(t_ref, a1_ref, b1_ref, w4_ref, m2_ref, mean2_ref, msq2_ref,
              s_acc, xs_acc):
    i = pl.program_id(0)

    @pl.when(i == 0)
    def _():
        s_acc[...] = jnp.zeros_like(s_acc)
        xs_acc[...] = jnp.zeros_like(xs_acc)

    x = jnp.maximum(t_ref[...] * a1_ref[...] + b1_ref[...], 0.0)
    s_acc[...] += lax.dot_general(x, x, (((0,), (0,)), ((), ())),
                                  preferred_element_type=jnp.float32)
    xs_acc[...] += jnp.sum(x, axis=0, keepdims=True)
    w4 = w4_ref[...]
    m2_ref[...] = lax.dot_general(x, w4, (((1,), (1,)), ((), ())),
                                  preferred_element_type=jnp.float32)

    @pl.when(i == pl.num_programs(0) - 1)
    def _():
        inv_e = 1.0 / E
        mean2_ref[...] = lax.dot_general(
            xs_acc[...], w4, (((1,), (1,)), ((), ())),
            preferred_element_type=jnp.float32) * inv_e
        ws = lax.dot_general(w4, s_acc[...], (((1,), (0,)), ((), ())),
                             preferred_element_type=jnp.float32)
        msq2_ref[...] = jnp.sum(ws * w4, axis=1)[None, :] * inv_e


NYCHUNK = N // KS


def _sc_scatter_body(m2_hbm, dst_hbm, s2_hbm, sh2_hbm,
                     y_hbm,
                     ixi0, ixi1, ixo0, ixo1, mi0, mi1, mo0, mo1,
                     s2_v, sh2_v, y_sh,
                     is0, is1, ss0, ss1):
    cid = lax.axis_index("c")
    sid = lax.axis_index("s")
    wid = sid * NC + cid

    pltpu.sync_copy(s2_hbm, s2_v)
    pltpu.sync_copy(sh2_hbm, sh2_v)

    for g in range(NG1):
        def zrow(r, carry):
            mi0[r, pl.ds(g * 16, 16)] = jnp.zeros((16,), jnp.float32)
            return carry
        lax.fori_loop(0, KS, zrow, 0)

    nz = (NYCHUNK - sid + NS - 1) // NS

    def zchunk(j, carry):
        pltpu.sync_copy(mi0, y_sh.at[pl.ds((sid + j * NS) * KS, KS)])
        return carry

    lax.fori_loop(0, nz, zchunk, 0)
    plsc.subcore_barrier()

    bufs = ((ixi0, ixo0, mi0, mo0, is0, ss0), (ixi1, ixo1, mi1, mo1, is1, ss1))

    def issue(c, b):
        ixi, _, mi, _, isem, _ = bufs[b]
        pltpu.async_copy(m2_hbm.at[pl.ds(wid * EW + c * KS, KS)], mi, isem)
        pltpu.async_copy(dst_hbm.at[pl.ds(wid * EW + c * KS, KS)], ixi.at[0],
                         isem)

    issue(0, 0)
    issue(1, 1)

    def outer(i2, carry_unused):
        for b in range(2):
            c = i2 * 2 + b

            @pl.when(c < NCS)
            def _():
                ixi, ixo, mi, mo, isem, ssem = bufs[b]
                pltpu.make_async_copy(
                    m2_hbm.at[pl.ds(wid * EW + c * KS, KS)], mi, isem).wait()
                pltpu.make_async_copy(
                    dst_hbm.at[pl.ds(wid * EW + c * KS, KS)], ixi.at[0],
                    isem).wait()

                @pl.when(c >= 2)
                def _():
                    pltpu.make_async_copy(mo, y_sh.at[ixo.at[0]], ssem).wait()

                def row(r, carry):
                    for g in range(NG1):
                        gsl = pl.ds(g * 16, 16)
                        mo[r, gsl] = jnp.maximum(
                            mi[r, gsl] * s2_v[gsl] + sh2_v[gsl], 0.0)
                    return carry

                lax.fori_loop(0, KS, row, 0)
                for gg in range(KS // 16):
                    ixo[0, pl.ds(gg * 16, 16)] = ixi[0, pl.ds(gg * 16, 16)]
                pltpu.async_copy(mo, y_sh.at[ixo.at[0]], ssem, add=True)

                @pl.when(c + 2 < NCS)
                def _():
                    issue(c + 2, b)
        return carry_unused

    lax.fori_loop(0, (NCS + 2) // 2, outer, 0)
    for c in (NCS - 2, NCS - 1):
        ixo = bufs[c % 2][1]
        mo = bufs[c % 2][3]
        ssem = bufs[c % 2][5]
        pltpu.make_async_copy(mo, y_sh.at[ixo.at[0]], ssem).wait()
    plsc.subcore_barrier()

    def wchunk(j, carry):
        cy = sid + j * NS
        pltpu.sync_copy(y_sh.at[pl.ds(cy * KS, KS)],
                        y_hbm.at[cid, pl.ds(cy * KS, KS)])
        return carry

    lax.fori_loop(0, nz, wchunk, 0)


def _mlp2_body(y2_ref, w5_ref, g3_ref, b3_ref, w6_ref, g4_ref, b4_ref,
               w7_ref, b7_ref, o_ref):
    dn = (((1,), (1,)), ((), ()))
    y = y2_ref[0] + y2_ref[1]
    z = lax.dot_general(y, w5_ref[...], dn, preferred_element_type=jnp.float32)
    mu = jnp.mean(z, axis=0, keepdims=True)
    v = jnp.mean((z - mu) ** 2, axis=0, keepdims=True)
    z = jnp.maximum((z - mu) * (g3_ref[...] * lax.rsqrt(v + EPS)) + b3_ref[...], 0.0)
    w = lax.dot_general(z, w6_ref[...], dn, preferred_element_type=jnp.float32)
    mu = jnp.mean(w, axis=0, keepdims=True)
    v = jnp.mean((w - mu) ** 2, axis=0, keepdims=True)
    w = jnp.maximum((w - mu) * (g4_ref[...] * lax.rsqrt(v + EPS)) + b4_ref[...], 0.0)
    o_ref[...] = lax.dot_general(w, w7_ref[...], dn,
                                 preferred_element_type=jnp.float32) + b7_ref[...]


def kernel(edge_rep, face_rep, edge_index, face_to_message_indicator,
           W1, W2, W3, g1, b1, W4, g2, b2, W5, g3, b3, W6, g4, b4, W7, b7):
    f32 = jnp.float32
    src = edge_index[0].astype(jnp.int32).reshape(NW, EW)
    dst = edge_index[1].astype(jnp.int32).reshape(NW, EW)
    f2m = face_to_message_indicator.astype(jnp.int32).reshape(NW, EW)
    dst1 = edge_index[1].astype(jnp.int32).reshape(E)

    A, B = pl.pallas_call(
        _proj2_body,
        out_shape=(jax.ShapeDtypeStruct((N, H2), f32),
                   jax.ShapeDtypeStruct((N, H2), f32)),
    )(edge_rep, W1, W2)
    C = pl.pallas_call(
        _proj1_body,
        out_shape=jax.ShapeDtypeStruct((F, H2), f32),
    )(face_rep, W3)

    mesh = plsc.VectorSubcoreMesh(core_axis_name="c", subcore_axis_name="s",
                                  num_cores=NC, num_subcores=NS)
    t, stats = pl.kernel(
        _sc_gather_body,
        out_type=(jax.ShapeDtypeStruct((E, H2), f32),
                  jax.ShapeDtypeStruct((NW, 2, H2), f32)),
        mesh=mesh,
        scratch_types=[
            pltpu.VMEM((EW,), jnp.int32),
            pltpu.VMEM((EW,), jnp.int32),
            pltpu.VMEM((EW,), jnp.int32),
            pltpu.VMEM((KG, H2), f32),
            pltpu.VMEM((KG, H2), f32),
            pltpu.VMEM((KG, H2), f32),
            pltpu.VMEM((KG, H2), f32),
            pltpu.VMEM((KG, H2), f32),
            pltpu.VMEM((KG, H2), f32),
            pltpu.VMEM((KG, H2), f32),
            pltpu.VMEM((KG, H2), f32),
            pltpu.VMEM((2, H2), f32),
            pltpu.SemaphoreType.DMA,
            pltpu.SemaphoreType.DMA,
            pltpu.SemaphoreType.DMA,
            pltpu.SemaphoreType.DMA,
        ],
    )(A, B, C, src, dst, f2m)

    st = jnp.sum(stats, axis=0)
    mu1 = st[0] / E
    var1 = st[1] / E - mu1 * mu1
    a1 = (g1 * lax.rsqrt(var1 + EPS))[None, :]
    b1p = (b1 - mu1 * a1[0])[None, :]

    BE = 3200
    m2, mean2, msq2 = pl.pallas_call(
        _bn1_body,
        grid=(E // BE,),
        in_specs=[
            pl.BlockSpec((BE, H2), lambda i: (i, 0)),
            pl.BlockSpec((1, H2), lambda i: (0, 0)),
            pl.BlockSpec((1, H2), lambda i: (0, 0)),
            pl.BlockSpec((H, H2), lambda i: (0, 0)),
        ],
        out_specs=[
            pl.BlockSpec((BE, H), lambda i: (i, 0)),
            pl.BlockSpec((1, H), lambda i: (0, 0)),
            pl.BlockSpec((1, H), lambda i: (0, 0)),
        ],
        out_shape=[
            jax.ShapeDtypeStruct((E, H), f32),
            jax.ShapeDtypeStruct((1, H), f32),
            jax.ShapeDtypeStruct((1, H), f32),
        ],
        scratch_shapes=[
            pltpu.VMEM((H2, H2), f32),
            pltpu.VMEM((1, H2), f32),
        ],
    )(t, a1, b1p, W4)

    var2 = msq2[0] - mean2[0] ** 2
    s2 = g2 * lax.rsqrt(var2 + EPS)
    sh2 = b2 - mean2[0] * s2

    y2 = pl.kernel(
        _sc_scatter_body,
        out_type=jax.ShapeDtypeStruct((NC, N, H), f32),
        mesh=mesh,
        scratch_types=[
            pltpu.VMEM((1, KS), jnp.int32),
            pltpu.VMEM((1, KS), jnp.int32),
            pltpu.VMEM((1, KS), jnp.int32),
            pltpu.VMEM((1, KS), jnp.int32),
            pltpu.VMEM((KS, H), f32),
            pltpu.VMEM((KS, H), f32),
            pltpu.VMEM((KS, H), f32),
            pltpu.VMEM((KS, H), f32),
            pltpu.VMEM((H,), f32),
            pltpu.VMEM((H,), f32),
            pltpu.VMEM_SHARED((N, H), f32),
            pltpu.SemaphoreType.DMA,
            pltpu.SemaphoreType.DMA,
            pltpu.SemaphoreType.DMA,
            pltpu.SemaphoreType.DMA,
        ],
    )(m2, dst1, s2, sh2)

    out = pl.pallas_call(
        _mlp2_body,
        out_shape=jax.ShapeDtypeStruct((N, H), f32),
    )(y2, W5, g3[None, :], b3[None, :], W6, g4[None, :], b4[None, :],
      W7, b7[None, :])
    return out


# speedup vs baseline: 3.3028x; 1.0010x over previous
import functools

import jax
import jax.numpy as jnp
from jax import lax
from jax.experimental import pallas as pl
from jax.experimental.pallas import tpu as pltpu
from jax.experimental.pallas import tpu_sc as plsc

N = 10000
F = 20000
E = 320000
H = 128
H2 = 256
EPS = 1e-5

NC = 2
NS = 16
NW = NC * NS
EW = E // NW

KG = 40
NCG = EW // KG
KS = 80
NCS = EW // KS
NG2 = H2 // 16
NG1 = H // 16


def _proj_body(e_ref, f_ref, w1_ref, w2_ref, w3_ref, o_ref):
    i = pl.program_id(0)
    dn = (((1,), (1,)), ((), ()))

    @pl.when(i == 0)
    def _():
        o_ref[...] = lax.dot_general(e_ref[...], w1_ref[...], dn,
                                     preferred_element_type=jnp.float32)

    @pl.when(i == 1)
    def _():
        o_ref[...] = lax.dot_general(e_ref[...], w2_ref[...], dn,
                                     preferred_element_type=jnp.float32)

    @pl.when(i == 2)
    def _():
        o_ref[...] = lax.dot_general(f_ref[0:N], w3_ref[...], dn,
                                     preferred_element_type=jnp.float32)

    @pl.when(i == 3)
    def _():
        o_ref[...] = lax.dot_general(f_ref[N:F], w3_ref[...], dn,
                                     preferred_element_type=jnp.float32)


def _sc_gather_body(tab_hbm, src_hbm, dst_hbm, f2m_hbm,
                    t_hbm, stats_hbm,
                    src_v, dst_v, f2m_v,
                    ar0, br0, cr0, tb0, ar1, br1, cr1, tb1,
                    acc, gs0, gs1, ws0, ws1):
    cid = lax.axis_index("c")
    sid = lax.axis_index("s")
    wid = sid * NC + cid

    pltpu.sync_copy(src_hbm.at[wid], src_v)
    pltpu.sync_copy(dst_hbm.at[wid], dst_v)
    pltpu.sync_copy(f2m_hbm.at[wid], f2m_v)

    for g in range(NG2):
        acc[0, pl.ds(g * 16, 16)] = jnp.zeros((16,), jnp.float32)
        acc[1, pl.ds(g * 16, 16)] = jnp.zeros((16,), jnp.float32)

    bufs = ((ar0, br0, cr0, tb0, gs0, ws0), (ar1, br1, cr1, tb1, gs1, ws1))

    def issue(c, b):
        ar, br, cr, _, gs, _ = bufs[b]
        sl = pl.ds(c * KG, KG)
        pltpu.async_copy(tab_hbm.at[src_v.at[sl]], ar, gs)
        pltpu.async_copy(tab_hbm.at[dst_v.at[sl]], br, gs)
        pltpu.async_copy(tab_hbm.at[f2m_v.at[sl]], cr, gs)

    issue(0, 0)
    issue(1, 1)

    zero16 = jnp.zeros((16,), jnp.float32)

    def outer(i2, carry_unused):
        for b in range(2):
            c = i2 * 2 + b
            ar, br, cr, tb, gs, ws = bufs[b]
            sl = pl.ds(c * KG, KG)
            pltpu.make_async_copy(tab_hbm.at[src_v.at[sl]], ar, gs).wait()
            pltpu.make_async_copy(tab_hbm.at[dst_v.at[sl]], br, gs).wait()
            pltpu.make_async_copy(tab_hbm.at[f2m_v.at[sl]], cr, gs).wait()

            @pl.when(c >= 2)
            def _():
                pltpu.make_async_copy(
                    tb, t_hbm.at[pl.ds(wid * EW + (c - 2) * KG, KG)], ws).wait()

            def row(r, carry):
                out_s = []
                out_q = []
                for g in range(NG2):
                    gsl = pl.ds(g * 16, 16)
                    t = ar[r, gsl] + br[r, gsl] + cr[r, gsl]
                    tb[r, gsl] = t
                    out_s.append(carry[g] + t)
                    out_q.append(carry[NG2 + g] + t * t)
                return tuple(out_s) + tuple(out_q)

            sums = lax.fori_loop(0, KG, row, (zero16,) * (2 * NG2))
            for g in range(NG2):
                gsl = pl.ds(g * 16, 16)
                acc[0, gsl] = acc[0, gsl] + sums[g]
                acc[1, gsl] = acc[1, gsl] + sums[NG2 + g]

            pltpu.async_copy(tb, t_hbm.at[pl.ds(wid * EW + c * KG, KG)], ws)

            @pl.when(c + 2 < NCG)
            def _():
                issue(c + 2, b)
        return carry_unused

    lax.fori_loop(0, NCG // 2, outer, 0)
    for b in range(2):
        tb = bufs[b][3]
        ws = bufs[b][5]
        pltpu.make_async_copy(
            tb, t_hbm.at[pl.ds(wid * EW + (NCG - 2 + b) * KG, KG)], ws).wait()
    pltpu.sync_copy(acc, stats_hbm.at[wid])


def _bn1_body(t_ref, a1_ref, b1_ref, w4_ref, m2_ref, mean2_ref, msq2_ref,
              s_acc, xs_acc):
    i = pl.program_id(0)

    @pl.when(i == 0)
    def _():
        s_acc[...] = jnp.zeros_like(s_acc)
        xs_acc[...] = jnp.zeros_like(xs_acc)

    x = jnp.maximum(t_ref[...] * a1_ref[...] + b1_ref[...], 0.0)
    xb = x.astype(jnp.bfloat16)
    s_acc[...] += lax.dot_general(xb, xb, (((0,), (0,)), ((), ())),
                                  preferred_element_type=jnp.float32)
    xs_acc[...] += jnp.sum(x, axis=0, keepdims=True)
    w4 = w4_ref[...]
    m2_ref[...] = lax.dot_general(xb, w4.astype(jnp.bfloat16),
                                  (((1,), (1,)), ((), ())),
                                  preferred_element_type=jnp.float32)

    @pl.when(i == pl.num_programs(0) - 1)
    def _():
        inv_e = 1.0 / E
        mean2_ref[...] = lax.dot_general(
            xs_acc[...], w4, (((1,), (1,)), ((), ())),
            preferred_element_type=jnp.float32) * inv_e
        ws = lax.dot_general(w4, s_acc[...], (((1,), (0,)), ((), ())),
                             preferred_element_type=jnp.float32)
        msq2_ref[...] = jnp.sum(ws * w4, axis=1)[None, :] * inv_e


NYCHUNK = N // KS


def _sc_scatter_body(m2_hbm, dst_hbm, s2_hbm, sh2_hbm,
                     y_hbm,
                     ixi0, ixi1, ixo0, ixo1, mi0, mi1, mo0, mo1,
                     s2_v, sh2_v, y_sh,
                     is0, is1, ss0, ss1):
    cid = lax.axis_index("c")
    sid = lax.axis_index("s")
    wid = sid * NC + cid

    pltpu.sync_copy(s2_hbm, s2_v)
    pltpu.sync_copy(sh2_hbm, sh2_v)

    for g in range(NG1):
        def zrow(r, carry):
            mi0[r, pl.ds(g * 16, 16)] = jnp.zeros((16,), jnp.float32)
            return carry
        lax.fori_loop(0, KS, zrow, 0)

    nz = (NYCHUNK - sid + NS - 1) // NS

    def zchunk(j, carry):
        pltpu.sync_copy(mi0, y_sh.at[pl.ds((sid + j * NS) * KS, KS)])
        return carry

    lax.fori_loop(0, nz, zchunk, 0)
    plsc.subcore_barrier()

    bufs = ((ixi0, ixo0, mi0, mo0, is0, ss0), (ixi1, ixo1, mi1, mo1, is1, ss1))

    def issue(c, b):
        ixi, _, mi, _, isem, _ = bufs[b]
        pltpu.async_copy(m2_hbm.at[pl.ds(wid * EW + c * KS, KS)], mi, isem)
        pltpu.async_copy(dst_hbm.at[pl.ds(wid * EW + c * KS, KS)], ixi.at[0],
                         isem)

    issue(0, 0)
    issue(1, 1)

    def outer(i2, carry_unused):
        for b in range(2):
            c = i2 * 2 + b

            @pl.when(c < NCS)
            def _():
                ixi, ixo, mi, mo, isem, ssem = bufs[b]
                pltpu.make_async_copy(
                    m2_hbm.at[pl.ds(wid * EW + c * KS, KS)], mi, isem).wait()
                pltpu.make_async_copy(
                    dst_hbm.at[pl.ds(wid * EW + c * KS, KS)], ixi.at[0],
                    isem).wait()

                @pl.when(c >= 2)
                def _():
                    pltpu.make_async_copy(mo, y_sh.at[ixo.at[0]], ssem).wait()

                def row(r, carry):
                    for g in range(NG1):
                        gsl = pl.ds(g * 16, 16)
                        mo[r, gsl] = jnp.maximum(
                            mi[r, gsl] * s2_v[gsl] + sh2_v[gsl], 0.0)
                    return carry

                lax.fori_loop(0, KS, row, 0)
                for gg in range(KS // 16):
                    ixo[0, pl.ds(gg * 16, 16)] = ixi[0, pl.ds(gg * 16, 16)]
                pltpu.async_copy(mo, y_sh.at[ixo.at[0]], ssem, add=True)

                @pl.when(c + 2 < NCS)
                def _():
                    issue(c + 2, b)
        return carry_unused

    lax.fori_loop(0, (NCS + 2) // 2, outer, 0)
    for c in (NCS - 2, NCS - 1):
        ixo = bufs[c % 2][1]
        mo = bufs[c % 2][3]
        ssem = bufs[c % 2][5]
        pltpu.make_async_copy(mo, y_sh.at[ixo.at[0]], ssem).wait()
    plsc.subcore_barrier()

    def wchunk(j, carry):
        cy = sid + j * NS
        pltpu.sync_copy(y_sh.at[pl.ds(cy * KS, KS)],
                        y_hbm.at[cid, pl.ds(cy * KS, KS)])
        return carry

    lax.fori_loop(0, nz, wchunk, 0)


def _mlp2_body(y2_ref, w5_ref, g3_ref, b3_ref, w6_ref, g4_ref, b4_ref,
               w7_ref, b7_ref, o_ref):
    dn = (((1,), (1,)), ((), ()))
    y = y2_ref[0] + y2_ref[1]
    z = lax.dot_general(y, w5_ref[...], dn, preferred_element_type=jnp.float32)
    mu = jnp.mean(z, axis=0, keepdims=True)
    v = jnp.mean((z - mu) ** 2, axis=0, keepdims=True)
    z = jnp.maximum((z - mu) * (g3_ref[...] * lax.rsqrt(v + EPS)) + b3_ref[...], 0.0)
    w = lax.dot_general(z, w6_ref[...], dn, preferred_element_type=jnp.float32)
    mu = jnp.mean(w, axis=0, keepdims=True)
    v = jnp.mean((w - mu) ** 2, axis=0, keepdims=True)
    w = jnp.maximum((w - mu) * (g4_ref[...] * lax.rsqrt(v + EPS)) + b4_ref[...], 0.0)
    o_ref[...] = lax.dot_general(w, w7_ref[...], dn,
                                 preferred_element_type=jnp.float32) + b7_ref[...]


def kernel(edge_rep, face_rep, edge_index, face_to_message_indicator,
           W1, W2, W3, g1, b1, W4, g2, b2, W5, g3, b3, W6, g4, b4, W7, b7):
    f32 = jnp.float32
    src = edge_index[0].astype(jnp.int32).reshape(NW, EW)
    dstt = (edge_index[1].astype(jnp.int32) + N).reshape(NW, EW)
    f2m = (face_to_message_indicator.astype(jnp.int32) + 2 * N).reshape(NW, EW)
    dst1 = edge_index[1].astype(jnp.int32).reshape(E)

    T = pl.pallas_call(
        _proj_body,
        grid=(4,),
        in_specs=[
            pl.BlockSpec((N, H), lambda i: (0, 0)),
            pl.BlockSpec((F, H), lambda i: (0, 0)),
            pl.BlockSpec((H2, H), lambda i: (0, 0)),
            pl.BlockSpec((H2, H), lambda i: (0, 0)),
            pl.BlockSpec((H2, H), lambda i: (0, 0)),
        ],
        out_specs=pl.BlockSpec((N, H2), lambda i: (i, 0)),
        out_shape=jax.ShapeDtypeStruct((2 * N + F, H2), f32),
    )(edge_rep, face_rep, W1, W2, W3)

    mesh = plsc.VectorSubcoreMesh(core_axis_name="c", subcore_axis_name="s",
                                  num_cores=NC, num_subcores=NS)
    t, stats = pl.kernel(
        _sc_gather_body,
        out_type=(jax.ShapeDtypeStruct((E, H2), f32),
                  jax.ShapeDtypeStruct((NW, 2, H2), f32)),
        mesh=mesh,
        scratch_types=[
            pltpu.VMEM((EW,), jnp.int32),
            pltpu.VMEM((EW,), jnp.int32),
            pltpu.VMEM((EW,), jnp.int32),
            pltpu.VMEM((KG, H2), f32),
            pltpu.VMEM((KG, H2), f32),
            pltpu.VMEM((KG, H2), f32),
            pltpu.VMEM((KG, H2), f32),
            pltpu.VMEM((KG, H2), f32),
            pltpu.VMEM((KG, H2), f32),
            pltpu.VMEM((KG, H2), f32),
            pltpu.VMEM((KG, H2), f32),
            pltpu.VMEM((2, H2), f32),
            pltpu.SemaphoreType.DMA,
            pltpu.SemaphoreType.DMA,
            pltpu.SemaphoreType.DMA,
            pltpu.SemaphoreType.DMA,
        ],
    )(T, src, dstt, f2m)

    st = jnp.sum(stats, axis=0)
    mu1 = st[0] / E
    var1 = st[1] / E - mu1 * mu1
    a1 = (g1 * lax.rsqrt(var1 + EPS))[None, :]
    b1p = (b1 - mu1 * a1[0])[None, :]

    BE = 3200
    m2, mean2, msq2 = pl.pallas_call(
        _bn1_body,
        grid=(E // BE,),
        in_specs=[
            pl.BlockSpec((BE, H2), lambda i: (i, 0)),
            pl.BlockSpec((1, H2), lambda i: (0, 0)),
            pl.BlockSpec((1, H2), lambda i: (0, 0)),
            pl.BlockSpec((H, H2), lambda i: (0, 0)),
        ],
        out_specs=[
            pl.BlockSpec((BE, H), lambda i: (i, 0)),
            pl.BlockSpec((1, H), lambda i: (0, 0)),
            pl.BlockSpec((1, H), lambda i: (0, 0)),
        ],
        out_shape=[
            jax.ShapeDtypeStruct((E, H), f32),
            jax.ShapeDtypeStruct((1, H), f32),
            jax.ShapeDtypeStruct((1, H), f32),
        ],
        scratch_shapes=[
            pltpu.VMEM((H2, H2), f32),
            pltpu.VMEM((1, H2), f32),
        ],
    )(t, a1, b1p, W4)

    var2 = msq2[0] - mean2[0] ** 2
    s2 = g2 * lax.rsqrt(var2 + EPS)
    sh2 = b2 - mean2[0] * s2

    y2 = pl.kernel(
        _sc_scatter_body,
        out_type=jax.ShapeDtypeStruct((NC, N, H), f32),
        mesh=mesh,
        scratch_types=[
            pltpu.VMEM((1, KS), jnp.int32),
            pltpu.VMEM((1, KS), jnp.int32),
            pltpu.VMEM((1, KS), jnp.int32),
            pltpu.VMEM((1, KS), jnp.int32),
            pltpu.VMEM((KS, H), f32),
            pltpu.VMEM((KS, H), f32),
            pltpu.VMEM((KS, H), f32),
            pltpu.VMEM((KS, H), f32),
            pltpu.VMEM((H,), f32),
            pltpu.VMEM((H,), f32),
            pltpu.VMEM_SHARED((N, H), f32),
            pltpu.SemaphoreType.DMA,
            pltpu.SemaphoreType.DMA,
            pltpu.SemaphoreType.DMA,
            pltpu.SemaphoreType.DMA,
        ],
    )(m2, dst1, s2, sh2)

    out = pl.pallas_call(
        _mlp2_body,
        out_shape=jax.ShapeDtypeStruct((N, H), f32),
    )(y2, W5, g3[None, :], b3[None, :], W6, g4[None, :], b4[None, :],
      W7, b7[None, :])
    return out


# speedup vs baseline: 3.3054x; 1.0008x over previous
import functools

import jax
import jax.numpy as jnp
from jax import lax
from jax.experimental import pallas as pl
from jax.experimental.pallas import tpu as pltpu
from jax.experimental.pallas import tpu_sc as plsc

N = 10000
F = 20000
E = 320000
H = 128
H2 = 256
EPS = 1e-5

NC = 2
NS = 16
NW = NC * NS
EW = E // NW

KG = 40
NCG = EW // KG
KS = 80
NCS = EW // KS
NG2 = H2 // 16
NG1 = H // 16


def _proj_body(e_ref, f_ref, w1_ref, w2_ref, w3_ref, o_ref):
    i = pl.program_id(0)
    dn = (((1,), (1,)), ((), ()))

    @pl.when(i == 0)
    def _():
        o_ref[...] = lax.dot_general(e_ref[...], w1_ref[...], dn,
                                     preferred_element_type=jnp.float32)

    @pl.when(i == 1)
    def _():
        o_ref[...] = lax.dot_general(e_ref[...], w2_ref[...], dn,
                                     preferred_element_type=jnp.float32)

    @pl.when(i == 2)
    def _():
        o_ref[...] = lax.dot_general(f_ref[0:N], w3_ref[...], dn,
                                     preferred_element_type=jnp.float32)

    @pl.when(i == 3)
    def _():
        o_ref[...] = lax.dot_general(f_ref[N:F], w3_ref[...], dn,
                                     preferred_element_type=jnp.float32)


def _sc_gather_body(tab_hbm, src_hbm, dst_hbm, f2m_hbm,
                    t_hbm, stats_hbm,
                    src_v, dst_v, f2m_v,
                    ar0, br0, cr0, tb0, ar1, br1, cr1, tb1,
                    acc, gs0, gs1, ws0, ws1):
    cid = lax.axis_index("c")
    sid = lax.axis_index("s")
    wid = sid * NC + cid

    pltpu.sync_copy(src_hbm.at[wid], src_v)
    pltpu.sync_copy(dst_hbm.at[wid], dst_v)
    pltpu.sync_copy(f2m_hbm.at[wid], f2m_v)

    for g in range(NG2):
        acc[0, pl.ds(g * 16, 16)] = jnp.zeros((16,), jnp.float32)
        acc[1, pl.ds(g * 16, 16)] = jnp.zeros((16,), jnp.float32)

    bufs = ((ar0, br0, cr0, tb0, gs0, ws0), (ar1, br1, cr1, tb1, gs1, ws1))

    def issue(c, b):
        ar, br, cr, _, gs, _ = bufs[b]
        sl = pl.ds(c * KG, KG)
        pltpu.async_copy(tab_hbm.at[src_v.at[sl]], ar, gs)
        pltpu.async_copy(tab_hbm.at[dst_v.at[sl]], br, gs)
        pltpu.async_copy(tab_hbm.at[f2m_v.at[sl]], cr, gs)

    issue(0, 0)
    issue(1, 1)

    zero16 = jnp.zeros((16,), jnp.float32)

    def outer(i2, carry_unused):
        for b in range(2):
            c = i2 * 2 + b
            ar, br, cr, tb, gs, ws = bufs[b]
            sl = pl.ds(c * KG, KG)
            pltpu.make_async_copy(tab_hbm.at[src_v.at[sl]], ar, gs).wait()
            pltpu.make_async_copy(tab_hbm.at[dst_v.at[sl]], br, gs).wait()
            pltpu.make_async_copy(tab_hbm.at[f2m_v.at[sl]], cr, gs).wait()

            @pl.when(c >= 2)
            def _():
                pltpu.make_async_copy(
                    tb, t_hbm.at[pl.ds(wid * EW + (c - 2) * KG, KG)], ws).wait()

            def row(r, carry):
                out_s = []
                out_q = []
                for g in range(NG2):
                    gsl = pl.ds(g * 16, 16)
                    t = ar[r, gsl] + br[r, gsl] + cr[r, gsl]
                    tb[r, gsl] = t
                    out_s.append(carry[g] + t)
                    out_q.append(carry[NG2 + g] + t * t)
                return tuple(out_s) + tuple(out_q)

            sums = lax.fori_loop(0, KG, row, (zero16,) * (2 * NG2))
            for g in range(NG2):
                gsl = pl.ds(g * 16, 16)
                acc[0, gsl] = acc[0, gsl] + sums[g]
                acc[1, gsl] = acc[1, gsl] + sums[NG2 + g]

            pltpu.async_copy(tb, t_hbm.at[pl.ds(wid * EW + c * KG, KG)], ws)

            @pl.when(c + 2 < NCG)
            def _():
                issue(c + 2, b)
        return carry_unused

    lax.fori_loop(0, NCG // 2, outer, 0)
    for b in range(2):
        tb = bufs[b][3]
        ws = bufs[b][5]
        pltpu.make_async_copy(
            tb, t_hbm.at[pl.ds(wid * EW + (NCG - 2 + b) * KG, KG)], ws).wait()
    pltpu.sync_copy(acc, stats_hbm.at[wid])


def _bn1_body(t_ref, a1_ref, b1_ref, w4_ref, m2_ref, mean2_ref, msq2_ref,
              s_acc, xs_acc):
    i = pl.program_id(0)

    @pl.when(i == 0)
    def _():
        s_acc[...] = jnp.zeros_like(s_acc)
        xs_acc[...] = jnp.zeros_like(xs_acc)

    x = jnp.maximum(t_ref[...] * a1_ref[...] + b1_ref[...], 0.0)
    s_acc[...] += lax.dot_general(x, x, (((0,), (0,)), ((), ())),
                                  preferred_element_type=jnp.float32)
    xs_acc[...] += jnp.sum(x, axis=0, keepdims=True)
    w4 = w4_ref[...]
    m2_ref[...] = lax.dot_general(x, w4, (((1,), (1,)), ((), ())),
                                  preferred_element_type=jnp.float32)

    @pl.when(i == pl.num_programs(0) - 1)
    def _():
        inv_e = 1.0 / E
        mean2_ref[...] = lax.dot_general(
            xs_acc[...], w4, (((1,), (1,)), ((), ())),
            preferred_element_type=jnp.float32) * inv_e
        ws = lax.dot_general(w4, s_acc[...], (((1,), (0,)), ((), ())),
                             preferred_element_type=jnp.float32)
        msq2_ref[...] = jnp.sum(ws * w4, axis=1)[None, :] * inv_e


NYCHUNK = N // KS


def _sc_scatter_body(m2_hbm, dst_hbm, s2_hbm, sh2_hbm,
                     y_hbm,
                     ixi0, ixi1, ixo0, ixo1, mi0, mi1, mo0, mo1,
                     s2_v, sh2_v, y_sh,
                     is0, is1, ss0, ss1):
    cid = lax.axis_index("c")
    sid = lax.axis_index("s")
    wid = sid * NC + cid

    pltpu.sync_copy(s2_hbm, s2_v)
    pltpu.sync_copy(sh2_hbm, sh2_v)

    for g in range(NG1):
        def zrow(r, carry):
            mi0[r, pl.ds(g * 16, 16)] = jnp.zeros((16,), jnp.float32)
            return carry
        lax.fori_loop(0, KS, zrow, 0)

    nz = (NYCHUNK - sid + NS - 1) // NS

    def zchunk(j, carry):
        pltpu.sync_copy(mi0, y_sh.at[pl.ds((sid + j * NS) * KS, KS)])
        return carry

    lax.fori_loop(0, nz, zchunk, 0)
    plsc.subcore_barrier()

    bufs = ((ixi0, ixo0, mi0, mo0, is0, ss0), (ixi1, ixo1, mi1, mo1, is1, ss1))

    def issue(c, b):
        ixi, _, mi, _, isem, _ = bufs[b]
        pltpu.async_copy(m2_hbm.at[pl.ds(wid * EW + c * KS, KS)], mi, isem)
        pltpu.async_copy(dst_hbm.at[pl.ds(wid * EW + c * KS, KS)], ixi.at[0],
                         isem)

    issue(0, 0)
    issue(1, 1)

    def outer(i2, carry_unused):
        for b in range(2):
            c = i2 * 2 + b

            @pl.when(c < NCS)
            def _():
                ixi, ixo, mi, mo, isem, ssem = bufs[b]
                pltpu.make_async_copy(
                    m2_hbm.at[pl.ds(wid * EW + c * KS, KS)], mi, isem).wait()
                pltpu.make_async_copy(
                    dst_hbm.at[pl.ds(wid * EW + c * KS, KS)], ixi.at[0],
                    isem).wait()

                @pl.when(c >= 2)
                def _():
                    pltpu.make_async_copy(mo, y_sh.at[ixo.at[0]], ssem).wait()

                def row(r, carry):
                    for g in range(NG1):
                        gsl = pl.ds(g * 16, 16)
                        mo[r, gsl] = jnp.maximum(
                            mi[r, gsl] * s2_v[gsl] + sh2_v[gsl], 0.0)
                    return carry

                lax.fori_loop(0, KS, row, 0)
                for gg in range(KS // 16):
                    ixo[0, pl.ds(gg * 16, 16)] = ixi[0, pl.ds(gg * 16, 16)]
                pltpu.async_copy(mo, y_sh.at[ixo.at[0]], ssem, add=True)

                @pl.when(c + 2 < NCS)
                def _():
                    issue(c + 2, b)
        return carry_unused

    lax.fori_loop(0, (NCS + 2) // 2, outer, 0)
    for c in (NCS - 2, NCS - 1):
        ixo = bufs[c % 2][1]
        mo = bufs[c % 2][3]
        ssem = bufs[c % 2][5]
        pltpu.make_async_copy(mo, y_sh.at[ixo.at[0]], ssem).wait()
    plsc.subcore_barrier()

    def wchunk(j, carry):
        cy = sid + j * NS
        pltpu.sync_copy(y_sh.at[pl.ds(cy * KS, KS)],
                        y_hbm.at[cid, pl.ds(cy * KS, KS)])
        return carry

    lax.fori_loop(0, nz, wchunk, 0)


def _mlp2_body(y2_ref, w5_ref, g3_ref, b3_ref, w6_ref, g4_ref, b4_ref,
               w7_ref, b7_ref, o_ref):
    dn = (((1,), (1,)), ((), ()))
    y = y2_ref[0] + y2_ref[1]
    z = lax.dot_general(y, w5_ref[...], dn, preferred_element_type=jnp.float32)
    mu = jnp.mean(z, axis=0, keepdims=True)
    v = jnp.mean((z - mu) ** 2, axis=0, keepdims=True)
    z = jnp.maximum((z - mu) * (g3_ref[...] * lax.rsqrt(v + EPS)) + b3_ref[...], 0.0)
    w = lax.dot_general(z, w6_ref[...], dn, preferred_element_type=jnp.float32)
    mu = jnp.mean(w, axis=0, keepdims=True)
    v = jnp.mean((w - mu) ** 2, axis=0, keepdims=True)
    w = jnp.maximum((w - mu) * (g4_ref[...] * lax.rsqrt(v + EPS)) + b4_ref[...], 0.0)
    o_ref[...] = lax.dot_general(w, w7_ref[...], dn,
                                 preferred_element_type=jnp.float32) + b7_ref[...]


def kernel(edge_rep, face_rep, edge_index, face_to_message_indicator,
           W1, W2, W3, g1, b1, W4, g2, b2, W5, g3, b3, W6, g4, b4, W7, b7):
    f32 = jnp.float32
    src = edge_index[0].astype(jnp.int32).reshape(NW, EW)
    dstt = (edge_index[1].astype(jnp.int32) + N).reshape(NW, EW)
    f2m = (face_to_message_indicator.astype(jnp.int32) + 2 * N).reshape(NW, EW)
    dst1 = edge_index[1].astype(jnp.int32).reshape(E)

    T = pl.pallas_call(
        _proj_body,
        grid=(4,),
        in_specs=[
            pl.BlockSpec((N, H), lambda i: (0, 0)),
            pl.BlockSpec((F, H), lambda i: (0, 0)),
            pl.BlockSpec((H2, H), lambda i: (0, 0)),
            pl.BlockSpec((H2, H), lambda i: (0, 0)),
            pl.BlockSpec((H2, H), lambda i: (0, 0)),
        ],
        out_specs=pl.BlockSpec((N, H2), lambda i: (i, 0)),
        out_shape=jax.ShapeDtypeStruct((2 * N + F, H2), f32),
    )(edge_rep, face_rep, W1, W2, W3)

    mesh = plsc.VectorSubcoreMesh(core_axis_name="c", subcore_axis_name="s",
                                  num_cores=NC, num_subcores=NS)
    t, stats = pl.kernel(
        _sc_gather_body,
        out_type=(jax.ShapeDtypeStruct((E, H2), f32),
                  jax.ShapeDtypeStruct((NW, 2, H2), f32)),
        mesh=mesh,
        scratch_types=[
            pltpu.VMEM((EW,), jnp.int32),
            pltpu.VMEM((EW,), jnp.int32),
            pltpu.VMEM((EW,), jnp.int32),
            pltpu.VMEM((KG, H2), f32),
            pltpu.VMEM((KG, H2), f32),
            pltpu.VMEM((KG, H2), f32),
            pltpu.VMEM((KG, H2), f32),
            pltpu.VMEM((KG, H2), f32),
            pltpu.VMEM((KG, H2), f32),
            pltpu.VMEM((KG, H2), f32),
            pltpu.VMEM((KG, H2), f32),
            pltpu.VMEM((2, H2), f32),
            pltpu.SemaphoreType.DMA,
            pltpu.SemaphoreType.DMA,
            pltpu.SemaphoreType.DMA,
            pltpu.SemaphoreType.DMA,
        ],
    )(T, src, dstt, f2m)

    st = jnp.sum(stats, axis=0)
    mu1 = st[0] / E
    var1 = st[1] / E - mu1 * mu1
    a1 = (g1 * lax.rsqrt(var1 + EPS))[None, :]
    b1p = (b1 - mu1 * a1[0])[None, :]

    BE = 3200
    m2, mean2, msq2 = pl.pallas_call(
        _bn1_body,
        grid=(E // BE,),
        in_specs=[
            pl.BlockSpec((BE, H2), lambda i: (i, 0)),
            pl.BlockSpec((1, H2), lambda i: (0, 0)),
            pl.BlockSpec((1, H2), lambda i: (0, 0)),
            pl.BlockSpec((H, H2), lambda i: (0, 0)),
        ],
        out_specs=[
            pl.BlockSpec((BE, H), lambda i: (i, 0)),
            pl.BlockSpec((1, H), lambda i: (0, 0)),
            pl.BlockSpec((1, H), lambda i: (0, 0)),
        ],
        out_shape=[
            jax.ShapeDtypeStruct((E, H), f32),
            jax.ShapeDtypeStruct((1, H), f32),
            jax.ShapeDtypeStruct((1, H), f32),
        ],
        scratch_shapes=[
            pltpu.VMEM((H2, H2), f32),
            pltpu.VMEM((1, H2), f32),
        ],
    )(t, a1, b1p, W4)

    var2 = msq2[0] - mean2[0] ** 2
    s2 = g2 * lax.rsqrt(var2 + EPS)
    sh2 = b2 - mean2[0] * s2

    y2 = pl.kernel(
        _sc_scatter_body,
        out_type=jax.ShapeDtypeStruct((NC, N, H), f32),
        mesh=mesh,
        scratch_types=[
            pltpu.VMEM((1, KS), jnp.int32),
            pltpu.VMEM((1, KS), jnp.int32),
            pltpu.VMEM((1, KS), jnp.int32),
            pltpu.VMEM((1, KS), jnp.int32),
            pltpu.VMEM((KS, H), f32),
            pltpu.VMEM((KS, H), f32),
            pltpu.VMEM((KS, H), f32),
            pltpu.VMEM((KS, H), f32),
            pltpu.VMEM((H,), f32),
            pltpu.VMEM((H,), f32),
            pltpu.VMEM_SHARED((N, H), f32),
            pltpu.SemaphoreType.DMA,
            pltpu.SemaphoreType.DMA,
            pltpu.SemaphoreType.DMA,
            pltpu.SemaphoreType.DMA,
        ],
    )(m2, dst1, s2, sh2)

    out = pl.pallas_call(
        _mlp2_body,
        out_shape=jax.ShapeDtypeStruct((N, H), f32),
    )(y2, W5, g3[None, :], b3[None, :], W6, g4[None, :], b4[None, :],
      W7, b7[None, :])
    return out
